# baseline (speedup 1.0000x reference)
import jax
import jax.numpy as jnp
from jax.experimental import pallas as pl
from jax.experimental.pallas import tpu as pltpu

N = 4096
BR = 256


def _prep_kernel(adj_ref, x_ref, w0_ref, d_ref, s0_ref):
    rowsum = 1.0 + jnp.sum(adj_ref[...], axis=1, keepdims=True)
    d = jax.lax.rsqrt(rowsum)
    d_ref[...] = d
    s0_ref[...] = d * jnp.dot(x_ref[...], w0_ref[...],
                              preferred_element_type=jnp.float32)


def _layer0_kernel(adj_ref, s0_ref, d_ref, w1_ref, b0_ref, s1_ref):
    i = pl.program_id(0)
    t = jnp.dot(adj_ref[...], s0_ref[...], preferred_element_type=jnp.float32)
    own = s0_ref[pl.ds(i * BR, BR), :]
    h0 = d_ref[...] * (t + own) + b0_ref[...]
    h0 = jnp.where(h0 >= 0, h0, 0.01 * h0)
    s1_ref[...] = d_ref[...] * jnp.dot(h0, w1_ref[...],
                                       preferred_element_type=jnp.float32)


def _layer1_kernel(adj_ref, s1_ref, d_ref, b1_ref, s_ref, wch_ref, wcs_ref,
                   bc_ref, h_ref, y_ref):
    i = pl.program_id(0)
    t = jnp.dot(adj_ref[...], s1_ref[...], preferred_element_type=jnp.float32)
    own = s1_ref[pl.ds(i * BR, BR), :]
    h = d_ref[...] * (t + own) + b1_ref[...]
    h_ref[...] = h
    logits = (jnp.dot(h, wch_ref[...], preferred_element_type=jnp.float32)
              + jnp.dot(s_ref[...], wcs_ref[...],
                        preferred_element_type=jnp.float32)
              + bc_ref[...])
    m = jnp.max(logits, axis=1, keepdims=True)
    e = jnp.exp(logits - m)
    y_ref[...] = e / jnp.sum(e, axis=1, keepdims=True)


def kernel(x, adj, S, W0, b0, W1, b1, Wc, bc):
    in_dim = x.shape[1]
    hid = W0.shape[1]
    f_dim = W1.shape[1]
    s_dim = S.shape[1]
    c_dim = Wc.shape[1]
    grid = (N // BR,)

    d, s0 = pl.pallas_call(
        _prep_kernel,
        grid=grid,
        in_specs=[
            pl.BlockSpec((BR, N), lambda i: (i, 0)),
            pl.BlockSpec((BR, in_dim), lambda i: (i, 0)),
            pl.BlockSpec((in_dim, hid), lambda i: (0, 0)),
        ],
        out_specs=[
            pl.BlockSpec((BR, 1), lambda i: (i, 0)),
            pl.BlockSpec((BR, hid), lambda i: (i, 0)),
        ],
        out_shape=[
            jax.ShapeDtypeStruct((N, 1), jnp.float32),
            jax.ShapeDtypeStruct((N, hid), jnp.float32),
        ],
        compiler_params=pltpu.CompilerParams(
            dimension_semantics=("arbitrary",)),
    )(adj, x, W0)

    s1 = pl.pallas_call(
        _layer0_kernel,
        grid=grid,
        in_specs=[
            pl.BlockSpec((BR, N), lambda i: (i, 0)),
            pl.BlockSpec((N, hid), lambda i: (0, 0)),
            pl.BlockSpec((BR, 1), lambda i: (i, 0)),
            pl.BlockSpec((hid, f_dim), lambda i: (0, 0)),
            pl.BlockSpec((1, hid), lambda i: (0, 0)),
        ],
        out_specs=pl.BlockSpec((BR, f_dim), lambda i: (i, 0)),
        out_shape=jax.ShapeDtypeStruct((N, f_dim), jnp.float32),
        compiler_params=pltpu.CompilerParams(
            dimension_semantics=("arbitrary",)),
    )(adj, s0, d, W1, b0.reshape(1, hid))

    h, y = pl.pallas_call(
        _layer1_kernel,
        grid=grid,
        in_specs=[
            pl.BlockSpec((BR, N), lambda i: (i, 0)),
            pl.BlockSpec((N, f_dim), lambda i: (0, 0)),
            pl.BlockSpec((BR, 1), lambda i: (i, 0)),
            pl.BlockSpec((1, f_dim), lambda i: (0, 0)),
            pl.BlockSpec((BR, s_dim), lambda i: (i, 0)),
            pl.BlockSpec((f_dim, c_dim), lambda i: (0, 0)),
            pl.BlockSpec((s_dim, c_dim), lambda i: (0, 0)),
            pl.BlockSpec((1, c_dim), lambda i: (0, 0)),
        ],
        out_specs=[
            pl.BlockSpec((BR, f_dim), lambda i: (i, 0)),
            pl.BlockSpec((BR, c_dim), lambda i: (i, 0)),
        ],
        out_shape=[
            jax.ShapeDtypeStruct((N, f_dim), jnp.float32),
            jax.ShapeDtypeStruct((N, c_dim), jnp.float32),
        ],
        compiler_params=pltpu.CompilerParams(
            dimension_semantics=("arbitrary",)),
    )(adj, s1, d, b1.reshape(1, f_dim), S, Wc[:f_dim], Wc[f_dim:],
      bc.reshape(1, c_dim))

    return (h, y)


# speedup vs baseline: 1.0220x; 1.0220x over previous
import jax
import jax.numpy as jnp
from jax.experimental import pallas as pl
from jax.experimental.pallas import tpu as pltpu

N = 4096
BR = 256


def _prep_kernel(adj_ref, x_ref, w0_ref, d_ref, s0_ref, adjb_ref):
    a = adj_ref[...]
    adjb_ref[...] = a.astype(jnp.bfloat16)
    rowsum = 1.0 + jnp.sum(a, axis=1, keepdims=True)
    d = jax.lax.rsqrt(rowsum)
    d_ref[...] = d
    s0 = d * jnp.dot(x_ref[...], w0_ref[...],
                     preferred_element_type=jnp.float32)
    s0_ref[...] = s0.astype(jnp.bfloat16)


def _layer0_kernel(adj_ref, s0_ref, d_ref, w1_ref, b0_ref, s1_ref):
    i = pl.program_id(0)
    t = jnp.dot(adj_ref[...], s0_ref[...], preferred_element_type=jnp.float32)
    own = s0_ref[pl.ds(i * BR, BR), :].astype(jnp.float32)
    h0 = d_ref[...] * (t + own) + b0_ref[...]
    h0 = jnp.where(h0 >= 0, h0, 0.01 * h0)
    s1 = d_ref[...] * jnp.dot(h0, w1_ref[...],
                              preferred_element_type=jnp.float32)
    s1_ref[...] = s1.astype(jnp.bfloat16)


def _layer1_kernel(adj_ref, s1_ref, d_ref, b1_ref, s_ref, wch_ref, wcs_ref,
                   bc_ref, h_ref, y_ref):
    i = pl.program_id(0)
    t = jnp.dot(adj_ref[...], s1_ref[...], preferred_element_type=jnp.float32)
    own = s1_ref[pl.ds(i * BR, BR), :].astype(jnp.float32)
    h = d_ref[...] * (t + own) + b1_ref[...]
    h_ref[...] = h
    logits = (jnp.dot(h, wch_ref[...], preferred_element_type=jnp.float32)
              + jnp.dot(s_ref[...], wcs_ref[...],
                        preferred_element_type=jnp.float32)
              + bc_ref[...])
    m = jnp.max(logits, axis=1, keepdims=True)
    e = jnp.exp(logits - m)
    y_ref[...] = e / jnp.sum(e, axis=1, keepdims=True)


def kernel(x, adj, S, W0, b0, W1, b1, Wc, bc):
    in_dim = x.shape[1]
    hid = W0.shape[1]
    f_dim = W1.shape[1]
    s_dim = S.shape[1]
    c_dim = Wc.shape[1]
    grid = (N // BR,)

    d, s0, adjb = pl.pallas_call(
        _prep_kernel,
        grid=grid,
        in_specs=[
            pl.BlockSpec((BR, N), lambda i: (i, 0)),
            pl.BlockSpec((BR, in_dim), lambda i: (i, 0)),
            pl.BlockSpec((in_dim, hid), lambda i: (0, 0)),
        ],
        out_specs=[
            pl.BlockSpec((BR, 1), lambda i: (i, 0)),
            pl.BlockSpec((BR, hid), lambda i: (i, 0)),
            pl.BlockSpec((BR, N), lambda i: (i, 0)),
        ],
        out_shape=[
            jax.ShapeDtypeStruct((N, 1), jnp.float32),
            jax.ShapeDtypeStruct((N, hid), jnp.bfloat16),
            jax.ShapeDtypeStruct((N, N), jnp.bfloat16),
        ],
        compiler_params=pltpu.CompilerParams(
            dimension_semantics=("arbitrary",)),
    )(adj, x, W0)

    s1 = pl.pallas_call(
        _layer0_kernel,
        grid=grid,
        in_specs=[
            pl.BlockSpec((BR, N), lambda i: (i, 0)),
            pl.BlockSpec((N, hid), lambda i: (0, 0)),
            pl.BlockSpec((BR, 1), lambda i: (i, 0)),
            pl.BlockSpec((hid, f_dim), lambda i: (0, 0)),
            pl.BlockSpec((1, hid), lambda i: (0, 0)),
        ],
        out_specs=pl.BlockSpec((BR, f_dim), lambda i: (i, 0)),
        out_shape=jax.ShapeDtypeStruct((N, f_dim), jnp.bfloat16),
        compiler_params=pltpu.CompilerParams(
            dimension_semantics=("arbitrary",)),
    )(adjb, s0, d, W1, b0.reshape(1, hid))

    h, y = pl.pallas_call(
        _layer1_kernel,
        grid=grid,
        in_specs=[
            pl.BlockSpec((BR, N), lambda i: (i, 0)),
            pl.BlockSpec((N, f_dim), lambda i: (0, 0)),
            pl.BlockSpec((BR, 1), lambda i: (i, 0)),
            pl.BlockSpec((1, f_dim), lambda i: (0, 0)),
            pl.BlockSpec((BR, s_dim), lambda i: (i, 0)),
            pl.BlockSpec((f_dim, c_dim), lambda i: (0, 0)),
            pl.BlockSpec((s_dim, c_dim), lambda i: (0, 0)),
            pl.BlockSpec((1, c_dim), lambda i: (0, 0)),
        ],
        out_specs=[
            pl.BlockSpec((BR, f_dim), lambda i: (i, 0)),
            pl.BlockSpec((BR, c_dim), lambda i: (i, 0)),
        ],
        out_shape=[
            jax.ShapeDtypeStruct((N, f_dim), jnp.float32),
            jax.ShapeDtypeStruct((N, c_dim), jnp.float32),
        ],
        compiler_params=pltpu.CompilerParams(
            dimension_semantics=("arbitrary",)),
    )(adjb, s1, d, b1.reshape(1, f_dim), S, Wc[:f_dim], Wc[f_dim:],
      bc.reshape(1, c_dim))

    return (h, y)


# speedup vs baseline: 1.0232x; 1.0012x over previous
import jax
import jax.numpy as jnp
from jax.experimental import pallas as pl
from jax.experimental.pallas import tpu as pltpu

N = 4096
BR = 256


def _prep_kernel(adj_ref, x_ref, w0_ref, d_ref, s0_ref, adjb_ref):
    a = adj_ref[...]
    adjb_ref[...] = a.astype(jnp.bfloat16)
    rowsum = 1.0 + jnp.sum(a, axis=1, keepdims=True)
    d = jax.lax.rsqrt(rowsum)
    d_ref[...] = d
    s0 = d * jnp.dot(x_ref[...], w0_ref[...],
                     preferred_element_type=jnp.float32)
    s0_ref[...] = s0.astype(jnp.bfloat16)


def _layer0_kernel(adj_ref, s0_ref, d_ref, w1_ref, b0_ref, s1_ref):
    i = pl.program_id(0)
    t = jnp.dot(adj_ref[...], s0_ref[...], preferred_element_type=jnp.float32)
    own = s0_ref[pl.ds(i * BR, BR), :].astype(jnp.float32)
    h0 = d_ref[...] * (t + own) + b0_ref[...]
    h0 = jnp.where(h0 >= 0, h0, 0.01 * h0)
    s1 = d_ref[...] * jnp.dot(h0, w1_ref[...],
                              preferred_element_type=jnp.float32)
    s1_ref[...] = s1.astype(jnp.bfloat16)


def _layer1_kernel(adj_ref, s1_ref, d_ref, b1_ref, s_ref, wch_ref, wcs_ref,
                   bc_ref, h_ref, y_ref):
    i = pl.program_id(0)
    t = jnp.dot(adj_ref[...], s1_ref[...], preferred_element_type=jnp.float32)
    own = s1_ref[pl.ds(i * BR, BR), :].astype(jnp.float32)
    h = d_ref[...] * (t + own) + b1_ref[...]
    h_ref[...] = h
    logits = (jnp.dot(h, wch_ref[...], preferred_element_type=jnp.float32)
              + jnp.dot(s_ref[...], wcs_ref[...],
                        preferred_element_type=jnp.float32)
              + bc_ref[...])
    m = jnp.max(logits, axis=1, keepdims=True)
    e = jnp.exp(logits - m)
    y_ref[...] = e / jnp.sum(e, axis=1, keepdims=True)


def kernel(x, adj, S, W0, b0, W1, b1, Wc, bc):
    in_dim = x.shape[1]
    hid = W0.shape[1]
    f_dim = W1.shape[1]
    s_dim = S.shape[1]
    c_dim = Wc.shape[1]
    grid = (N // BR,)

    d, s0, adjb = pl.pallas_call(
        _prep_kernel,
        grid=grid,
        in_specs=[
            pl.BlockSpec((BR, N), lambda i: (i, 0)),
            pl.BlockSpec((BR, in_dim), lambda i: (i, 0)),
            pl.BlockSpec((in_dim, hid), lambda i: (0, 0)),
        ],
        out_specs=[
            pl.BlockSpec((BR, 1), lambda i: (i, 0)),
            pl.BlockSpec((BR, hid), lambda i: (i, 0)),
            pl.BlockSpec((BR, N), lambda i: (i, 0)),
        ],
        out_shape=[
            jax.ShapeDtypeStruct((N, 1), jnp.float32),
            jax.ShapeDtypeStruct((N, hid), jnp.bfloat16),
            jax.ShapeDtypeStruct((N, N), jnp.bfloat16),
        ],
        compiler_params=pltpu.CompilerParams(
            dimension_semantics=("parallel",)),
    )(adj, x, W0)

    s1 = pl.pallas_call(
        _layer0_kernel,
        grid=grid,
        in_specs=[
            pl.BlockSpec((BR, N), lambda i: (i, 0)),
            pl.BlockSpec((N, hid), lambda i: (0, 0)),
            pl.BlockSpec((BR, 1), lambda i: (i, 0)),
            pl.BlockSpec((hid, f_dim), lambda i: (0, 0)),
            pl.BlockSpec((1, hid), lambda i: (0, 0)),
        ],
        out_specs=pl.BlockSpec((BR, f_dim), lambda i: (i, 0)),
        out_shape=jax.ShapeDtypeStruct((N, f_dim), jnp.bfloat16),
        compiler_params=pltpu.CompilerParams(
            dimension_semantics=("parallel",)),
    )(adjb, s0, d, W1, b0.reshape(1, hid))

    h, y = pl.pallas_call(
        _layer1_kernel,
        grid=grid,
        in_specs=[
            pl.BlockSpec((BR, N), lambda i: (i, 0)),
            pl.BlockSpec((N, f_dim), lambda i: (0, 0)),
            pl.BlockSpec((BR, 1), lambda i: (i, 0)),
            pl.BlockSpec((1, f_dim), lambda i: (0, 0)),
            pl.BlockSpec((BR, s_dim), lambda i: (i, 0)),
            pl.BlockSpec((f_dim, c_dim), lambda i: (0, 0)),
            pl.BlockSpec((s_dim, c_dim), lambda i: (0, 0)),
            pl.BlockSpec((1, c_dim), lambda i: (0, 0)),
        ],
        out_specs=[
            pl.BlockSpec((BR, f_dim), lambda i: (i, 0)),
            pl.BlockSpec((BR, c_dim), lambda i: (i, 0)),
        ],
        out_shape=[
            jax.ShapeDtypeStruct((N, f_dim), jnp.float32),
            jax.ShapeDtypeStruct((N, c_dim), jnp.float32),
        ],
        compiler_params=pltpu.CompilerParams(
            dimension_semantics=("parallel",)),
    )(adjb, s1, d, b1.reshape(1, f_dim), S, Wc[:f_dim], Wc[f_dim:],
      bc.reshape(1, c_dim))

    return (h, y)


# speedup vs baseline: 1.4395x; 1.4068x over previous
import jax
import jax.numpy as jnp
from jax.experimental import pallas as pl
from jax.experimental.pallas import tpu as pltpu

N = 4096
BA = 256
BB = 512
NA = N // BA
NB = N // BB
P_B = NA
P_C = NA + NB


def _gcn_kernel(adj_ref, x_ref, w0_ref, w1_ref, b0_ref, b1_ref, s_in_ref,
                wch_ref, wcs_ref, bc_ref, h_ref, y_ref,
                adjb_scr, s0_scr, s1_scr, d_scr):
    i = pl.program_id(0)

    @pl.when(i < P_B)
    def _phase_a():
        a = adj_ref[...]
        adjb_scr[pl.ds(i * BA, BA), :] = a.astype(jnp.bfloat16)
        d = jax.lax.rsqrt(1.0 + jnp.sum(a, axis=1, keepdims=True))
        d_scr[pl.ds(i * BA, BA), :] = d
        s0 = d * jnp.dot(x_ref[...], w0_ref[...],
                         preferred_element_type=jnp.float32)
        s0_scr[pl.ds(i * BA, BA), :] = s0.astype(jnp.bfloat16)

    @pl.when(jnp.logical_and(i >= P_B, i < P_C))
    def _phase_b():
        r = (i - P_B) * BB
        t = jnp.dot(adjb_scr[pl.ds(r, BB), :], s0_scr[...],
                    preferred_element_type=jnp.float32)
        own = s0_scr[pl.ds(r, BB), :].astype(jnp.float32)
        h0 = d_scr[pl.ds(r, BB), :] * (t + own) + b0_ref[...]
        h0 = jnp.where(h0 >= 0, h0, 0.01 * h0)
        s1 = d_scr[pl.ds(r, BB), :] * jnp.dot(
            h0, w1_ref[...], preferred_element_type=jnp.float32)
        s1_scr[pl.ds(r, BB), :] = s1.astype(jnp.bfloat16)

    @pl.when(i >= P_C)
    def _phase_c():
        r = (i - P_C) * BB
        t = jnp.dot(adjb_scr[pl.ds(r, BB), :], s1_scr[...],
                    preferred_element_type=jnp.float32)
        own = s1_scr[pl.ds(r, BB), :].astype(jnp.float32)
        h = d_scr[pl.ds(r, BB), :] * (t + own) + b1_ref[...]
        h_ref[...] = h
        logits = (jnp.dot(h, wch_ref[...], preferred_element_type=jnp.float32)
                  + jnp.dot(s_in_ref[...], wcs_ref[...],
                            preferred_element_type=jnp.float32)
                  + bc_ref[...])
        m = jnp.max(logits, axis=1, keepdims=True)
        e = jnp.exp(logits - m)
        y_ref[...] = e / jnp.sum(e, axis=1, keepdims=True)


def kernel(x, adj, S, W0, b0, W1, b1, Wc, bc):
    in_dim = x.shape[1]
    hid = W0.shape[1]
    f_dim = W1.shape[1]
    s_dim = S.shape[1]
    c_dim = Wc.shape[1]

    def a_map(i):
        return (jnp.minimum(i, NA - 1), 0)

    def c_map(i):
        return (jnp.clip(i - P_C, 0, NB - 1), 0)

    h, y = pl.pallas_call(
        _gcn_kernel,
        grid=(NA + NB + NB,),
        in_specs=[
            pl.BlockSpec((BA, N), a_map),
            pl.BlockSpec((BA, in_dim), a_map),
            pl.BlockSpec((in_dim, hid), lambda i: (0, 0)),
            pl.BlockSpec((hid, f_dim), lambda i: (0, 0)),
            pl.BlockSpec((1, hid), lambda i: (0, 0)),
            pl.BlockSpec((1, f_dim), lambda i: (0, 0)),
            pl.BlockSpec((BB, s_dim), c_map),
            pl.BlockSpec((f_dim, c_dim), lambda i: (0, 0)),
            pl.BlockSpec((s_dim, c_dim), lambda i: (0, 0)),
            pl.BlockSpec((1, c_dim), lambda i: (0, 0)),
        ],
        out_specs=[
            pl.BlockSpec((BB, f_dim), c_map),
            pl.BlockSpec((BB, c_dim), c_map),
        ],
        out_shape=[
            jax.ShapeDtypeStruct((N, f_dim), jnp.float32),
            jax.ShapeDtypeStruct((N, c_dim), jnp.float32),
        ],
        scratch_shapes=[
            pltpu.VMEM((N, N), jnp.bfloat16),
            pltpu.VMEM((N, hid), jnp.bfloat16),
            pltpu.VMEM((N, f_dim), jnp.bfloat16),
            pltpu.VMEM((N, 1), jnp.float32),
        ],
        compiler_params=pltpu.CompilerParams(
            dimension_semantics=("arbitrary",)),
    )(adj, x, W0, W1, b0.reshape(1, hid), b1.reshape(1, f_dim), S,
      Wc[:f_dim], Wc[f_dim:], bc.reshape(1, c_dim))

    return (h, y)


# speedup vs baseline: 1.5028x; 1.0440x over previous
import jax
import jax.numpy as jnp
from jax.experimental import pallas as pl
from jax.experimental.pallas import tpu as pltpu

N = 4096
BA = 256
BB = 1024
NA = N // BA
NB = N // BB
P_B = NA
P_C = NA + NB


def _gcn_kernel(adj_ref, x_ref, w0_ref, w1_ref, b0_ref, b1_ref, s_in_ref,
                wch_ref, wcs_ref, bc_ref, h_ref, y_ref,
                adjb_scr, s0_scr, s1_scr, d_scr):
    i = pl.program_id(0)

    @pl.when(i < P_B)
    def _phase_a():
        a = adj_ref[...]
        adjb_scr[pl.ds(i * BA, BA), :] = a.astype(jnp.bfloat16)
        d = jax.lax.rsqrt(1.0 + jnp.sum(a, axis=1, keepdims=True))
        d_scr[pl.ds(i * BA, BA), :] = d
        s0 = d * jnp.dot(x_ref[...], w0_ref[...],
                         preferred_element_type=jnp.float32)
        s0_scr[pl.ds(i * BA, BA), :] = s0.astype(jnp.bfloat16)

    @pl.when(jnp.logical_and(i >= P_B, i < P_C))
    def _phase_b():
        r = (i - P_B) * BB
        t = jnp.dot(adjb_scr[pl.ds(r, BB), :], s0_scr[...],
                    preferred_element_type=jnp.float32)
        own = s0_scr[pl.ds(r, BB), :].astype(jnp.float32)
        h0 = d_scr[pl.ds(r, BB), :] * (t + own) + b0_ref[...]
        h0 = jnp.where(h0 >= 0, h0, 0.01 * h0)
        s1 = d_scr[pl.ds(r, BB), :] * jnp.dot(
            h0, w1_ref[...], preferred_element_type=jnp.float32)
        s1_scr[pl.ds(r, BB), :] = s1.astype(jnp.bfloat16)

    @pl.when(i >= P_C)
    def _phase_c():
        r = (i - P_C) * BB
        t = jnp.dot(adjb_scr[pl.ds(r, BB), :], s1_scr[...],
                    preferred_element_type=jnp.float32)
        own = s1_scr[pl.ds(r, BB), :].astype(jnp.float32)
        h = d_scr[pl.ds(r, BB), :] * (t + own) + b1_ref[...]
        h_ref[...] = h
        logits = (jnp.dot(h, wch_ref[...], preferred_element_type=jnp.float32)
                  + jnp.dot(s_in_ref[...], wcs_ref[...],
                            preferred_element_type=jnp.float32)
                  + bc_ref[...])
        m = jnp.max(logits, axis=1, keepdims=True)
        e = jnp.exp(logits - m)
        y_ref[...] = e / jnp.sum(e, axis=1, keepdims=True)


def kernel(x, adj, S, W0, b0, W1, b1, Wc, bc):
    in_dim = x.shape[1]
    hid = W0.shape[1]
    f_dim = W1.shape[1]
    s_dim = S.shape[1]
    c_dim = Wc.shape[1]

    def a_map(i):
        return (jnp.minimum(i, NA - 1), 0)

    def c_map(i):
        return (jnp.clip(i - P_C, 0, NB - 1), 0)

    h, y = pl.pallas_call(
        _gcn_kernel,
        grid=(NA + NB + NB,),
        in_specs=[
            pl.BlockSpec((BA, N), a_map),
            pl.BlockSpec((BA, in_dim), a_map),
            pl.BlockSpec((in_dim, hid), lambda i: (0, 0)),
            pl.BlockSpec((hid, f_dim), lambda i: (0, 0)),
            pl.BlockSpec((1, hid), lambda i: (0, 0)),
            pl.BlockSpec((1, f_dim), lambda i: (0, 0)),
            pl.BlockSpec((BB, s_dim), c_map),
            pl.BlockSpec((f_dim, c_dim), lambda i: (0, 0)),
            pl.BlockSpec((s_dim, c_dim), lambda i: (0, 0)),
            pl.BlockSpec((1, c_dim), lambda i: (0, 0)),
        ],
        out_specs=[
            pl.BlockSpec((BB, f_dim), c_map),
            pl.BlockSpec((BB, c_dim), c_map),
        ],
        out_shape=[
            jax.ShapeDtypeStruct((N, f_dim), jnp.float32),
            jax.ShapeDtypeStruct((N, c_dim), jnp.float32),
        ],
        scratch_shapes=[
            pltpu.VMEM((N, N), jnp.bfloat16),
            pltpu.VMEM((N, hid), jnp.bfloat16),
            pltpu.VMEM((N, f_dim), jnp.bfloat16),
            pltpu.VMEM((N, 1), jnp.float32),
        ],
        compiler_params=pltpu.CompilerParams(
            dimension_semantics=("arbitrary",)),
    )(adj, x, W0, W1, b0.reshape(1, hid), b1.reshape(1, f_dim), S,
      Wc[:f_dim], Wc[f_dim:], bc.reshape(1, c_dim))

    return (h, y)
